# Initial kernel scaffold; baseline (speedup 1.0000x reference)
import functools

import jax
import jax.numpy as jnp
from jax import lax
from jax.experimental import pallas as pl
from jax.experimental.pallas import tpu as pltpu
from jax.experimental.pallas import tpu_sc as plsc

D = 128
E = 16384
LANES = 16
NUM_WORKERS = 32
EDGES_PER_WORKER = E // NUM_WORKERS
CHUNK = 128
NUM_CHUNKS = EDGES_PER_WORKER // CHUNK
GROUPS = CHUNK // LANES


def _build():
    mesh = plsc.VectorSubcoreMesh(core_axis_name="c", subcore_axis_name="s")

    @functools.partial(
        pl.kernel,
        mesh=mesh,
        out_type=jax.ShapeDtypeStruct((E,), jnp.float32),
        scratch_types=[
            pltpu.VMEM((NUM_CHUNKS, CHUNK), jnp.int32),
            pltpu.VMEM((NUM_CHUNKS, CHUNK), jnp.int32),
            pltpu.VMEM((CHUNK, D), jnp.float32),
            pltpu.VMEM((CHUNK, D), jnp.float32),
            pltpu.VMEM((LANES, LANES), jnp.float32),
            pltpu.VMEM((EDGES_PER_WORKER,), jnp.float32),
            pltpu.SemaphoreType.DMA,
            pltpu.SemaphoreType.DMA,
        ],
    )
    def k(src_hbm, dst_hbm, table_hbm, out_hbm,
          sidx, didx, srows, drows, tbuf, outv, sem0, sem1):
        wid = lax.axis_index("s") * 2 + lax.axis_index("c")
        base_row = wid * NUM_CHUNKS
        pltpu.sync_copy(src_hbm.at[pl.ds(base_row, NUM_CHUNKS)], sidx)
        pltpu.sync_copy(dst_hbm.at[pl.ds(base_row, NUM_CHUNKS)], didx)
        lane_iota = lax.iota(jnp.int32, LANES)

        @pl.loop(0, NUM_CHUNKS)
        def _chunk(j):
            cp0 = pltpu.async_copy(table_hbm.at[sidx.at[j]], srows, sem0)
            cp1 = pltpu.async_copy(table_hbm.at[didx.at[j]], drows, sem1)
            cp0.wait()
            cp1.wait()

            @pl.loop(0, GROUPS)
            def _group(g):
                ebase = g * LANES
                for jj in range(LANES):
                    e = ebase + jj
                    acc = srows[e, pl.ds(0, LANES)] * drows[e, pl.ds(0, LANES)]
                    for c in range(1, D // LANES):
                        acc = acc + (srows[e, pl.ds(c * LANES, LANES)]
                                     * drows[e, pl.ds(c * LANES, LANES)])
                    col = jnp.full((LANES,), jj, jnp.int32)
                    plsc.store_scatter(tbuf, [lane_iota, col], acc)
                tot = tbuf[0]
                for l in range(1, LANES):
                    tot = tot + tbuf[l]
                outv[pl.ds(j * CHUNK + ebase, LANES)] = tot

        pltpu.sync_copy(outv, out_hbm.at[pl.ds(wid * EDGES_PER_WORKER,
                                               EDGES_PER_WORKER)])

    return k


_sc_dot = _build()


def kernel(edge_index, table):
    ei = edge_index.astype(jnp.int32)
    src = ei[0].reshape(NUM_WORKERS * NUM_CHUNKS, CHUNK)
    dst = ei[1].reshape(NUM_WORKERS * NUM_CHUNKS, CHUNK)
    return _sc_dot(src, dst, table)


# speedup vs baseline: 1.1823x; 1.1823x over previous
import dataclasses
import functools

import jax
import jax.numpy as jnp
from jax import lax
from jax.experimental import pallas as pl
from jax.experimental.pallas import tpu as pltpu
from jax.experimental.pallas import tpu_sc as plsc

D = 128
E = 16384
LANES = 16
NUM_WORKERS = 32
EDGES_PER_WORKER = E // NUM_WORKERS
CHUNK = 128
NUM_CHUNKS = EDGES_PER_WORKER // CHUNK
GROUPS = CHUNK // LANES


def _build():
    mesh = plsc.VectorSubcoreMesh(core_axis_name="c", subcore_axis_name="s")
    cp = pltpu.CompilerParams()
    if "needs_layout_passes" in pltpu.CompilerParams.__dataclass_fields__:
        cp = dataclasses.replace(cp, needs_layout_passes=False)

    @functools.partial(
        pl.kernel,
        mesh=mesh,
        compiler_params=cp,
        out_type=jax.ShapeDtypeStruct((E,), jnp.float32),
        scratch_types=[
            pltpu.VMEM((NUM_CHUNKS, CHUNK), jnp.int32),
            pltpu.VMEM((NUM_CHUNKS, CHUNK), jnp.int32),
            pltpu.VMEM((CHUNK, D), jnp.float32),
            pltpu.VMEM((CHUNK, D), jnp.float32),
            pltpu.VMEM((LANES, LANES), jnp.float32),
            pltpu.VMEM((EDGES_PER_WORKER,), jnp.float32),
            pltpu.SemaphoreType.DMA,
            pltpu.SemaphoreType.DMA,
        ],
    )
    def k(src_hbm, dst_hbm, table_hbm, out_hbm,
          sidx, didx, srows, drows, tbuf, outv, sem0, sem1):
        wid = lax.axis_index("s") * 2 + lax.axis_index("c")
        base_row = wid * NUM_CHUNKS
        pltpu.sync_copy(src_hbm.at[pl.ds(base_row, NUM_CHUNKS)], sidx)
        pltpu.sync_copy(dst_hbm.at[pl.ds(base_row, NUM_CHUNKS)], didx)
        lane_iota = lax.iota(jnp.int32, LANES)

        @pl.loop(0, NUM_CHUNKS)
        def _chunk(j):
            cp0 = pltpu.async_copy(table_hbm.at[sidx.at[j]], srows, sem0)
            cp1 = pltpu.async_copy(table_hbm.at[didx.at[j]], drows, sem1)
            cp0.wait()
            cp1.wait()

            @pl.loop(0, GROUPS)
            def _group(g):
                ebase = g * LANES
                for jj in range(LANES):
                    e = ebase + jj
                    acc = srows[e, pl.ds(0, LANES)] * drows[e, pl.ds(0, LANES)]
                    for c in range(1, D // LANES):
                        acc = acc + (srows[e, pl.ds(c * LANES, LANES)]
                                     * drows[e, pl.ds(c * LANES, LANES)])
                    col = jnp.full((LANES,), jj, jnp.int32)
                    plsc.store_scatter(tbuf, [lane_iota, col], acc)
                tot = tbuf[0]
                for l in range(1, LANES):
                    tot = tot + tbuf[l]
                outv[pl.ds(j * CHUNK + ebase, LANES)] = tot

        pltpu.sync_copy(outv, out_hbm.at[pl.ds(wid * EDGES_PER_WORKER,
                                               EDGES_PER_WORKER)])

    return k


_sc_dot = _build()


def kernel(edge_index, table):
    ei = edge_index.astype(jnp.int32)
    src = ei[0].reshape(NUM_WORKERS * NUM_CHUNKS, CHUNK)
    dst = ei[1].reshape(NUM_WORKERS * NUM_CHUNKS, CHUNK)
    return _sc_dot(src, dst, table)


# speedup vs baseline: 1.2504x; 1.0577x over previous
import dataclasses
import functools

import jax
import jax.numpy as jnp
from jax import lax
from jax.experimental import pallas as pl
from jax.experimental.pallas import tpu as pltpu
from jax.experimental.pallas import tpu_sc as plsc

D = 128
E = 16384
LANES = 16
NUM_WORKERS = 32
EDGES_PER_WORKER = E // NUM_WORKERS
CHUNK = 128
NUM_CHUNKS = EDGES_PER_WORKER // CHUNK
GROUPS = CHUNK // LANES


def _build():
    mesh = plsc.VectorSubcoreMesh(core_axis_name="c", subcore_axis_name="s")
    cp = pltpu.CompilerParams()
    if "needs_layout_passes" in pltpu.CompilerParams.__dataclass_fields__:
        cp = dataclasses.replace(cp, needs_layout_passes=False)

    @functools.partial(
        pl.kernel,
        mesh=mesh,
        compiler_params=cp,
        out_type=jax.ShapeDtypeStruct((E,), jnp.float32),
        scratch_types=[
            pltpu.VMEM((NUM_CHUNKS, CHUNK), jnp.int32),
            pltpu.VMEM((NUM_CHUNKS, CHUNK), jnp.int32),
            pltpu.VMEM((CHUNK, D), jnp.float32),
            pltpu.VMEM((CHUNK, D), jnp.float32),
            pltpu.VMEM((CHUNK, D), jnp.float32),
            pltpu.VMEM((CHUNK, D), jnp.float32),
            pltpu.VMEM((LANES, LANES), jnp.float32),
            pltpu.VMEM((EDGES_PER_WORKER,), jnp.float32),
            pltpu.SemaphoreType.DMA,
            pltpu.SemaphoreType.DMA,
        ],
    )
    def k(src_hbm, dst_hbm, table_hbm, out_hbm,
          sidx, didx, srows0, drows0, srows1, drows1, tbuf, outv, sem0, sem1):
        wid = lax.axis_index("s") * 2 + lax.axis_index("c")
        base_row = wid * NUM_CHUNKS
        pltpu.sync_copy(src_hbm.at[pl.ds(base_row, NUM_CHUNKS)], sidx)
        pltpu.sync_copy(dst_hbm.at[pl.ds(base_row, NUM_CHUNKS)], didx)
        lane_iota = lax.iota(jnp.int32, LANES)
        sbufs = (srows0, srows1)
        dbufs = (drows0, drows1)
        sems = (sem0, sem1)

        def fire(j):
            b = j % 2
            c0 = pltpu.async_copy(table_hbm.at[sidx.at[j]], sbufs[b], sems[b])
            c1 = pltpu.async_copy(table_hbm.at[didx.at[j]], dbufs[b], sems[b])
            return c0, c1

        def compute(j):
            srows, drows = sbufs[j % 2], dbufs[j % 2]

            @pl.loop(0, GROUPS)
            def _group(g):
                ebase = g * LANES
                for jj in range(LANES):
                    e = ebase + jj
                    acc = srows[e, pl.ds(0, LANES)] * drows[e, pl.ds(0, LANES)]
                    for c in range(1, D // LANES):
                        acc = acc + (srows[e, pl.ds(c * LANES, LANES)]
                                     * drows[e, pl.ds(c * LANES, LANES)])
                    col = jnp.full((LANES,), jj, jnp.int32)
                    plsc.store_scatter(tbuf, [lane_iota, col], acc)
                tot = tbuf[0]
                for l in range(1, LANES):
                    tot = tot + tbuf[l]
                outv[pl.ds(j * CHUNK + ebase, LANES)] = tot

        pending = fire(0)
        for j in range(NUM_CHUNKS):
            nxt = fire(j + 1) if j + 1 < NUM_CHUNKS else None
            pending[0].wait()
            pending[1].wait()
            compute(j)
            pending = nxt

        pltpu.sync_copy(outv, out_hbm.at[pl.ds(wid * EDGES_PER_WORKER,
                                               EDGES_PER_WORKER)])

    return k


_sc_dot = _build()


def kernel(edge_index, table):
    ei = edge_index.astype(jnp.int32)
    src = ei[0].reshape(NUM_WORKERS * NUM_CHUNKS, CHUNK)
    dst = ei[1].reshape(NUM_WORKERS * NUM_CHUNKS, CHUNK)
    return _sc_dot(src, dst, table)
